# baseline (speedup 1.0000x reference)
import jax
import jax.numpy as jnp
from jax.experimental import pallas as pl
from jax.experimental.pallas import tpu as pltpu

NUM_CLASSES_ = 1000


def _onehot_block(x_ref, o_ref):
    i = pl.program_id(0)
    b = x_ref.shape[1]
    ids = jax.lax.broadcasted_iota(jnp.int32, (NUM_CLASSES_, b), 0)
    xv = x_ref[pl.ds(i, 1), :]
    o_ref[...] = (ids == xv).astype(o_ref.dtype)[None]


def kernel(x):
    out_dtype = jnp.zeros((), jnp.int64).dtype
    b, s = x.shape
    xt = jnp.transpose(x).astype(jnp.int32)
    out_t = pl.pallas_call(
        _onehot_block,
        grid=(s,),
        in_specs=[pl.BlockSpec(memory_space=pltpu.MemorySpace.VMEM)],
        out_specs=pl.BlockSpec((1, NUM_CLASSES_, b), lambda i: (i, 0, 0)),
        out_shape=jax.ShapeDtypeStruct((s, NUM_CLASSES_, b), out_dtype),
    )(xt)
    return jnp.transpose(out_t, (2, 0, 1))
